# Initial kernel scaffold; baseline (speedup 1.0000x reference)
import jax
import jax.numpy as jnp
from jax.experimental import pallas as pl


def kernel(user_emb, item_emb, social_user_w1, user_w1, item_w1, w_r1_wT, w_r1_b, alpha, adj_s, adj_u, adj_i):
    raise NotImplementedError("write your pallas kernel here")


# speedup vs baseline: 1.5963x; 1.5963x over previous
import functools
import math

import jax
import jax.numpy as jnp
from jax.experimental import pallas as pl
from jax.experimental.pallas import tpu as pltpu

_VMEM_LIMIT = 40 * 1024 * 1024
_TK = 2048
_GROUPS = 5


def _prelu(y, alpha):
    return jnp.where(y > 0, y, alpha * y)


def _gsum(y):
    tm, _ = y.shape
    ng = tm // _GROUPS
    j = jax.lax.broadcasted_iota(jnp.int32, (ng, tm), 0)
    r = jax.lax.broadcasted_iota(jnp.int32, (ng, tm), 1)
    sel = (r // _GROUPS == j).astype(jnp.float32)
    return jnp.dot(sel, y, preferred_element_type=jnp.float32)


def _accumulate(adj_ref, x_ref, acc_ref, *, tk):
    k = pl.program_id(1)

    @pl.when(k == 0)
    def _():
        acc_ref[...] = jnp.zeros_like(acc_ref)

    xb = x_ref[pl.ds(pl.multiple_of(pl.program_id(1) * tk, tk), tk), :]
    acc_ref[...] += jnp.dot(adj_ref[...], xb,
                            preferred_element_type=jnp.float32)


def _l1_body(*refs, tk, emit_y, emit_sum):
    it = iter(refs)
    adj_ref, x_ref, w_ref, alpha_ref = next(it), next(it), next(it), next(it)
    y_ref = next(it) if emit_y else None
    yw_ref = next(it)
    s_ref = next(it) if emit_sum else None
    acc_ref = next(it)

    _accumulate(adj_ref, x_ref, acc_ref, tk=tk)

    @pl.when(pl.program_id(1) == pl.num_programs(1) - 1)
    def _():
        y = _prelu(acc_ref[...], alpha_ref[...])
        yw_ref[...] = jnp.dot(
            y.astype(jnp.bfloat16), w_ref[...],
            preferred_element_type=jnp.float32).astype(jnp.bfloat16)
        if emit_y:
            y_ref[...] = y
        if emit_sum:
            s_ref[...] = _gsum(y)


def _l2_social_body(adj_ref, x_ref, s0_ref, wr_ref, b_ref, alpha_ref,
                    o_ref, acc_ref, *, tk):
    _accumulate(adj_ref, x_ref, acc_ref, tk=tk)

    @pl.when(pl.program_id(1) == pl.num_programs(1) - 1)
    def _():
        e1 = _prelu(acc_ref[...], alpha_ref[...])
        s = jnp.concatenate([s0_ref[...], _gsum(e1)], axis=-1) * 0.25
        o_ref[...] = (jnp.dot(s.astype(jnp.bfloat16), wr_ref[...],
                              preferred_element_type=jnp.float32)
                      + b_ref[...]).astype(jnp.bfloat16)


def _l2_out_body(adj_ref, x_ref, left_ref, alpha_ref, o_ref, acc_ref, *,
                 tk, group, scale):
    _accumulate(adj_ref, x_ref, acc_ref, tk=tk)

    @pl.when(pl.program_id(1) == pl.num_programs(1) - 1)
    def _():
        y = _prelu(acc_ref[...], alpha_ref[...])
        if group:
            y = _gsum(y)
        o = jnp.concatenate([left_ref[...], y], axis=-1)
        if scale != 1.0:
            o = o * scale
        o_ref[...] = o


def _bcast_spec(shape):
    return pl.BlockSpec(shape, lambda i, k: (0, 0))


def _row_spec(rows, cols):
    return pl.BlockSpec((rows, cols), lambda i, k: (i, 0))


def _spmm(body, adj, x, extras, extra_specs, out_shapes, out_specs, *, tm):
    M, K = adj.shape
    D = x.shape[1]
    grid = (M // tm, K // _TK)
    in_specs = [pl.BlockSpec((tm, _TK), lambda i, k: (i, k)),
                _bcast_spec((K, D))] + extra_specs
    flops = 2 * M * K * D
    bytes_accessed = M * K * 2 + K * D * 2 + sum(
        math.prod(s.shape) * s.dtype.itemsize for s in out_shapes)
    return pl.pallas_call(
        body,
        out_shape=tuple(out_shapes),
        grid=grid,
        in_specs=in_specs,
        out_specs=tuple(out_specs),
        scratch_shapes=[pltpu.VMEM((tm, D), jnp.float32)],
        compiler_params=pltpu.CompilerParams(
            dimension_semantics=("parallel", "arbitrary"),
            vmem_limit_bytes=_VMEM_LIMIT),
        cost_estimate=pl.CostEstimate(flops=int(flops), transcendentals=0,
                                      bytes_accessed=int(bytes_accessed)),
    )(adj, x, *extras)


def kernel(user_emb, item_emb, social_user_w1, user_w1, item_w1, w_r1_wT,
           w_r1_b, alpha, adj_s, adj_u, adj_i):
    bf16 = jnp.bfloat16
    n_users, n_items, D = 2048, 2048, 64
    Ms = adj_s.shape[0]
    alpha_row = jnp.broadcast_to(alpha, (1, D)).astype(jnp.float32)

    tm_g = 640
    ng = tm_g // _GROUPS

    ego_s, sum0 = _spmm(
        functools.partial(_l1_body, tk=_TK, emit_y=False, emit_sum=True),
        adj_s, user_emb.astype(bf16),
        [social_user_w1.astype(bf16), alpha_row],
        [_bcast_spec((D, D)), _bcast_spec((1, D))],
        [jax.ShapeDtypeStruct((Ms, D), bf16),
         jax.ShapeDtypeStruct((n_users, D), jnp.float32)],
        [_row_spec(tm_g, D), _row_spec(ng, D)],
        tm=tm_g)

    user_vec = _spmm(
        functools.partial(_l2_social_body, tk=_TK),
        adj_s, ego_s,
        [sum0, w_r1_wT.astype(bf16), w_r1_b[None, :], alpha_row],
        [_row_spec(ng, D), _bcast_spec((2 * D, D)), _bcast_spec((1, D)),
         _bcast_spec((1, D))],
        [jax.ShapeDtypeStruct((n_users, D), bf16)],
        [_row_spec(ng, D)],
        tm=tm_g)[0]

    ego = jnp.concatenate([user_vec, item_emb.astype(bf16)], axis=0)
    tm_u = 512
    u0, u0w = _spmm(
        functools.partial(_l1_body, tk=_TK, emit_y=True, emit_sum=False),
        adj_u, ego,
        [user_w1.astype(bf16), alpha_row],
        [_bcast_spec((D, D)), _bcast_spec((1, D))],
        [jax.ShapeDtypeStruct((n_users, D), jnp.float32),
         jax.ShapeDtypeStruct((n_users, D), bf16)],
        [_row_spec(tm_u, D), _row_spec(tm_u, D)],
        tm=tm_u)
    i0w, sum_i0 = _spmm(
        functools.partial(_l1_body, tk=_TK, emit_y=False, emit_sum=True),
        adj_i, ego,
        [item_w1.astype(bf16), alpha_row],
        [_bcast_spec((D, D)), _bcast_spec((1, D))],
        [jax.ShapeDtypeStruct((Ms, D), bf16),
         jax.ShapeDtypeStruct((n_items, D), jnp.float32)],
        [_row_spec(tm_g, D), _row_spec(ng, D)],
        tm=tm_g)

    ego1 = jnp.concatenate([u0w, i0w], axis=0)
    user_embedding = _spmm(
        functools.partial(_l2_out_body, tk=_TK, group=False, scale=1.0),
        adj_u, ego1,
        [u0, alpha_row],
        [_row_spec(tm_u, D), _bcast_spec((1, D))],
        [jax.ShapeDtypeStruct((n_users, 2 * D), jnp.float32)],
        [_row_spec(tm_u, 2 * D)],
        tm=tm_u)[0]
    item_embedding = _spmm(
        functools.partial(_l2_out_body, tk=_TK, group=True, scale=1.0 / 5.0),
        adj_i, ego1,
        [sum_i0, alpha_row],
        [_row_spec(ng, D), _bcast_spec((1, D))],
        [jax.ShapeDtypeStruct((n_items, 2 * D), jnp.float32)],
        [_row_spec(ng, 2 * D)],
        tm=tm_g)[0]

    return user_embedding, item_embedding


# speedup vs baseline: 2.3753x; 1.4880x over previous
import functools
import math

import jax
import jax.numpy as jnp
from jax.experimental import pallas as pl
from jax.experimental.pallas import tpu as pltpu

_VMEM_LIMIT = 40 * 1024 * 1024
_GROUPS = 5


def _prelu(y, alpha):
    return jnp.where(y > 0, y, alpha * y)


def _gsum(y):
    tm, _ = y.shape
    ng = tm // _GROUPS
    j = jax.lax.broadcasted_iota(jnp.int32, (ng, tm), 0)
    r = jax.lax.broadcasted_iota(jnp.int32, (ng, tm), 1)
    sel = (r // _GROUPS == j).astype(jnp.float32)
    return jnp.dot(sel, y, preferred_element_type=jnp.float32)


def _accumulate(adj_ref, x_ref, acc_ref, *, tk):
    k = pl.program_id(1)

    @pl.when(k == 0)
    def _():
        acc_ref[...] = jnp.zeros_like(acc_ref)

    xb = x_ref[pl.ds(pl.multiple_of(pl.program_id(1) * tk, tk), tk), :]
    acc_ref[...] += jnp.dot(adj_ref[...], xb,
                            preferred_element_type=jnp.float32)


def _l1_body(*refs, tk, emit_y, emit_sum):
    it = iter(refs)
    adj_ref, x_ref, w_ref, alpha_ref = next(it), next(it), next(it), next(it)
    y_ref = next(it) if emit_y else None
    yw_ref = next(it)
    s_ref = next(it) if emit_sum else None
    acc_ref = next(it)

    _accumulate(adj_ref, x_ref, acc_ref, tk=tk)

    @pl.when(pl.program_id(1) == pl.num_programs(1) - 1)
    def _():
        y = _prelu(acc_ref[...], alpha_ref[...])
        yw_ref[...] = jnp.dot(
            y.astype(jnp.bfloat16), w_ref[...],
            preferred_element_type=jnp.float32).astype(jnp.bfloat16)
        if emit_y:
            y_ref[...] = y
        if emit_sum:
            s_ref[...] = _gsum(y)


def _l2_social_body(adj_ref, x_ref, s0_ref, wr_ref, b_ref, alpha_ref,
                    o_ref, acc_ref, *, tk):
    _accumulate(adj_ref, x_ref, acc_ref, tk=tk)

    @pl.when(pl.program_id(1) == pl.num_programs(1) - 1)
    def _():
        e1 = _prelu(acc_ref[...], alpha_ref[...])
        s = jnp.concatenate([s0_ref[...], _gsum(e1)], axis=-1) * 0.25
        o_ref[...] = (jnp.dot(s.astype(jnp.bfloat16), wr_ref[...],
                              preferred_element_type=jnp.float32)
                      + b_ref[...]).astype(jnp.bfloat16)


def _l2_out_body(adj_ref, x_ref, left_ref, alpha_ref, o_ref, acc_ref, *,
                 tk, group, scale):
    _accumulate(adj_ref, x_ref, acc_ref, tk=tk)

    @pl.when(pl.program_id(1) == pl.num_programs(1) - 1)
    def _():
        y = _prelu(acc_ref[...], alpha_ref[...])
        if group:
            y = _gsum(y)
        o = jnp.concatenate([left_ref[...], y], axis=-1)
        if scale != 1.0:
            o = o * scale
        o_ref[...] = o


def _bcast_spec(shape):
    return pl.BlockSpec(shape, lambda i, k: (0, 0))


def _row_spec(rows, cols):
    return pl.BlockSpec((rows, cols), lambda i, k: (i, 0))


def _spmm(body, adj, x, extras, extra_specs, out_shapes, out_specs, *, tm, tk):
    M, K = adj.shape
    D = x.shape[1]
    grid = (M // tm, K // tk)
    in_specs = [pl.BlockSpec((tm, tk), lambda i, k: (i, k)),
                _bcast_spec((K, D))] + extra_specs
    flops = 2 * M * K * D
    bytes_accessed = M * K * 2 + K * D * 2 + sum(
        math.prod(s.shape) * s.dtype.itemsize for s in out_shapes)
    return pl.pallas_call(
        body,
        out_shape=tuple(out_shapes),
        grid=grid,
        in_specs=in_specs,
        out_specs=tuple(out_specs),
        scratch_shapes=[pltpu.VMEM((tm, D), jnp.float32)],
        compiler_params=pltpu.CompilerParams(
            dimension_semantics=("parallel", "arbitrary"),
            vmem_limit_bytes=_VMEM_LIMIT),
        cost_estimate=pl.CostEstimate(flops=int(flops), transcendentals=0,
                                      bytes_accessed=int(bytes_accessed)),
    )(adj, x, *extras)


def kernel(user_emb, item_emb, social_user_w1, user_w1, item_w1, w_r1_wT,
           w_r1_b, alpha, adj_s, adj_u, adj_i):
    bf16 = jnp.bfloat16
    n_users, n_items, D = 2048, 2048, 64
    Ms = adj_s.shape[0]
    alpha_row = jnp.broadcast_to(alpha, (1, D)).astype(jnp.float32)

    tm_g = 1280
    ng = tm_g // _GROUPS

    ego_s, sum0 = _spmm(
        functools.partial(_l1_body, tk=5120, emit_y=False, emit_sum=True),
        adj_s, user_emb.astype(bf16),
        [social_user_w1.astype(bf16), alpha_row],
        [_bcast_spec((D, D)), _bcast_spec((1, D))],
        [jax.ShapeDtypeStruct((Ms, D), bf16),
         jax.ShapeDtypeStruct((n_users, D), jnp.float32)],
        [_row_spec(tm_g, D), _row_spec(ng, D)],
        tm=tm_g, tk=5120)

    user_vec = _spmm(
        functools.partial(_l2_social_body, tk=5120),
        adj_s, ego_s,
        [sum0, w_r1_wT.astype(bf16), w_r1_b[None, :], alpha_row],
        [_row_spec(ng, D), _bcast_spec((2 * D, D)), _bcast_spec((1, D)),
         _bcast_spec((1, D))],
        [jax.ShapeDtypeStruct((n_users, D), bf16)],
        [_row_spec(ng, D)],
        tm=tm_g, tk=5120)[0]

    ego = jnp.concatenate([user_vec, item_emb.astype(bf16)], axis=0)
    tm_u = 512
    u0, u0w = _spmm(
        functools.partial(_l1_body, tk=6144, emit_y=True, emit_sum=False),
        adj_u, ego,
        [user_w1.astype(bf16), alpha_row],
        [_bcast_spec((D, D)), _bcast_spec((1, D))],
        [jax.ShapeDtypeStruct((n_users, D), jnp.float32),
         jax.ShapeDtypeStruct((n_users, D), bf16)],
        [_row_spec(tm_u, D), _row_spec(tm_u, D)],
        tm=tm_u, tk=6144)
    i0w, sum_i0 = _spmm(
        functools.partial(_l1_body, tk=6144, emit_y=False, emit_sum=True),
        adj_i, ego,
        [item_w1.astype(bf16), alpha_row],
        [_bcast_spec((D, D)), _bcast_spec((1, D))],
        [jax.ShapeDtypeStruct((Ms, D), bf16),
         jax.ShapeDtypeStruct((n_items, D), jnp.float32)],
        [_row_spec(tm_g, D), _row_spec(ng, D)],
        tm=tm_g, tk=6144)

    ego1 = jnp.concatenate([u0w, i0w], axis=0)
    user_embedding = _spmm(
        functools.partial(_l2_out_body, tk=6144, group=False, scale=1.0),
        adj_u, ego1,
        [u0, alpha_row],
        [_row_spec(tm_u, D), _bcast_spec((1, D))],
        [jax.ShapeDtypeStruct((n_users, 2 * D), jnp.float32)],
        [_row_spec(tm_u, 2 * D)],
        tm=tm_u, tk=6144)[0]
    item_embedding = _spmm(
        functools.partial(_l2_out_body, tk=6144, group=True, scale=1.0 / 5.0),
        adj_i, ego1,
        [sum_i0, alpha_row],
        [_row_spec(ng, D), _bcast_spec((1, D))],
        [jax.ShapeDtypeStruct((n_items, 2 * D), jnp.float32)],
        [_row_spec(ng, 2 * D)],
        tm=tm_g, tk=6144)[0]

    return user_embedding, item_embedding


# speedup vs baseline: 2.4431x; 1.0286x over previous
import functools
import math

import jax
import jax.numpy as jnp
from jax.experimental import pallas as pl
from jax.experimental.pallas import tpu as pltpu

_VMEM_LIMIT = 40 * 1024 * 1024
_GROUPS = 5


def _prelu(y, alpha):
    return jnp.where(y > 0, y, alpha * y)


def _gsum(y):
    tm, _ = y.shape
    ng = tm // _GROUPS
    j = jax.lax.broadcasted_iota(jnp.int32, (ng, tm), 0)
    r = jax.lax.broadcasted_iota(jnp.int32, (ng, tm), 1)
    sel = (r // _GROUPS == j).astype(jnp.float32)
    return jnp.dot(sel, y, preferred_element_type=jnp.float32)


def _accumulate(adj_ref, x_ref, acc_ref, *, tk):
    k = pl.program_id(1)

    @pl.when(k == 0)
    def _():
        acc_ref[...] = jnp.zeros_like(acc_ref)

    xb = x_ref[pl.ds(pl.multiple_of(pl.program_id(1) * tk, tk), tk), :]
    acc_ref[...] += jnp.dot(adj_ref[...], xb,
                            preferred_element_type=jnp.float32)


def _l1_body(*refs, tk, emit_y, emit_sum):
    it = iter(refs)
    adj_ref, x_ref, w_ref, alpha_ref = next(it), next(it), next(it), next(it)
    y_ref = next(it) if emit_y else None
    yw_ref = next(it)
    s_ref = next(it) if emit_sum else None
    acc_ref = next(it)

    _accumulate(adj_ref, x_ref, acc_ref, tk=tk)

    @pl.when(pl.program_id(1) == pl.num_programs(1) - 1)
    def _():
        y = _prelu(acc_ref[...], alpha_ref[...])
        yw_ref[...] = jnp.dot(
            y.astype(jnp.bfloat16), w_ref[...],
            preferred_element_type=jnp.float32).astype(jnp.bfloat16)
        if emit_y:
            y_ref[...] = y
        if emit_sum:
            s_ref[...] = _gsum(y)


def _l2_social_body(adj_ref, x_ref, s0_ref, wr_ref, b_ref, alpha_ref,
                    o_ref, acc_ref, *, tk):
    _accumulate(adj_ref, x_ref, acc_ref, tk=tk)

    @pl.when(pl.program_id(1) == pl.num_programs(1) - 1)
    def _():
        e1 = _prelu(acc_ref[...], alpha_ref[...])
        s = jnp.concatenate([s0_ref[...], _gsum(e1)], axis=-1) * 0.25
        o_ref[...] = (jnp.dot(s.astype(jnp.bfloat16), wr_ref[...],
                              preferred_element_type=jnp.float32)
                      + b_ref[...]).astype(jnp.bfloat16)


def _l2_out_body(adj_ref, x_ref, left_ref, alpha_ref, o_ref, acc_ref, *,
                 tk, group, scale):
    _accumulate(adj_ref, x_ref, acc_ref, tk=tk)

    @pl.when(pl.program_id(1) == pl.num_programs(1) - 1)
    def _():
        y = _prelu(acc_ref[...], alpha_ref[...])
        if group:
            y = _gsum(y)
        o = jnp.concatenate([left_ref[...], y], axis=-1)
        if scale != 1.0:
            o = o * scale
        o_ref[...] = o


def _bcast_spec(shape):
    return pl.BlockSpec(shape, lambda i, k: (0, 0))


def _row_spec(rows, cols):
    return pl.BlockSpec((rows, cols), lambda i, k: (i, 0))


def _spmm(body, adj, x, extras, extra_specs, out_shapes, out_specs, *, tm, tk):
    M, K = adj.shape
    D = x.shape[1]
    grid = (M // tm, K // tk)
    in_specs = [pl.BlockSpec((tm, tk), lambda i, k: (i, k)),
                _bcast_spec((K, D))] + extra_specs
    flops = 2 * M * K * D
    bytes_accessed = M * K * 2 + K * D * 2 + sum(
        math.prod(s.shape) * s.dtype.itemsize for s in out_shapes)
    return pl.pallas_call(
        body,
        out_shape=tuple(out_shapes),
        grid=grid,
        in_specs=in_specs,
        out_specs=tuple(out_specs),
        scratch_shapes=[pltpu.VMEM((tm, D), jnp.float32)],
        compiler_params=pltpu.CompilerParams(
            dimension_semantics=("parallel", "arbitrary"),
            vmem_limit_bytes=_VMEM_LIMIT),
        cost_estimate=pl.CostEstimate(flops=int(flops), transcendentals=0,
                                      bytes_accessed=int(bytes_accessed)),
    )(adj, x, *extras)


def kernel(user_emb, item_emb, social_user_w1, user_w1, item_w1, w_r1_wT,
           w_r1_b, alpha, adj_s, adj_u, adj_i):
    bf16 = jnp.bfloat16
    n_users, n_items, D = 2048, 2048, 64
    Ms = adj_s.shape[0]
    alpha_row = jnp.broadcast_to(alpha, (1, D)).astype(jnp.float32)

    tm_g = 640
    ng = tm_g // _GROUPS

    ego_s, sum0 = _spmm(
        functools.partial(_l1_body, tk=10240, emit_y=False, emit_sum=True),
        adj_s, user_emb.astype(bf16),
        [social_user_w1.astype(bf16), alpha_row],
        [_bcast_spec((D, D)), _bcast_spec((1, D))],
        [jax.ShapeDtypeStruct((Ms, D), bf16),
         jax.ShapeDtypeStruct((n_users, D), jnp.float32)],
        [_row_spec(tm_g, D), _row_spec(ng, D)],
        tm=tm_g, tk=10240)

    user_vec = _spmm(
        functools.partial(_l2_social_body, tk=10240),
        adj_s, ego_s,
        [sum0, w_r1_wT.astype(bf16), w_r1_b[None, :], alpha_row],
        [_row_spec(ng, D), _bcast_spec((2 * D, D)), _bcast_spec((1, D)),
         _bcast_spec((1, D))],
        [jax.ShapeDtypeStruct((n_users, D), bf16)],
        [_row_spec(ng, D)],
        tm=tm_g, tk=10240)[0]

    ego = jnp.concatenate([user_vec, item_emb.astype(bf16)], axis=0)
    tm_u = 512
    u0, u0w = _spmm(
        functools.partial(_l1_body, tk=12288, emit_y=True, emit_sum=False),
        adj_u, ego,
        [user_w1.astype(bf16), alpha_row],
        [_bcast_spec((D, D)), _bcast_spec((1, D))],
        [jax.ShapeDtypeStruct((n_users, D), jnp.float32),
         jax.ShapeDtypeStruct((n_users, D), bf16)],
        [_row_spec(tm_u, D), _row_spec(tm_u, D)],
        tm=tm_u, tk=12288)
    i0w, sum_i0 = _spmm(
        functools.partial(_l1_body, tk=12288, emit_y=False, emit_sum=True),
        adj_i, ego,
        [item_w1.astype(bf16), alpha_row],
        [_bcast_spec((D, D)), _bcast_spec((1, D))],
        [jax.ShapeDtypeStruct((Ms, D), bf16),
         jax.ShapeDtypeStruct((n_items, D), jnp.float32)],
        [_row_spec(tm_g, D), _row_spec(ng, D)],
        tm=tm_g, tk=12288)

    ego1 = jnp.concatenate([u0w, i0w], axis=0)
    user_embedding = _spmm(
        functools.partial(_l2_out_body, tk=12288, group=False, scale=1.0),
        adj_u, ego1,
        [u0, alpha_row],
        [_row_spec(tm_u, D), _bcast_spec((1, D))],
        [jax.ShapeDtypeStruct((n_users, 2 * D), jnp.float32)],
        [_row_spec(tm_u, 2 * D)],
        tm=tm_u, tk=12288)[0]
    item_embedding = _spmm(
        functools.partial(_l2_out_body, tk=12288, group=True, scale=1.0 / 5.0),
        adj_i, ego1,
        [sum_i0, alpha_row],
        [_row_spec(ng, D), _bcast_spec((1, D))],
        [jax.ShapeDtypeStruct((n_items, 2 * D), jnp.float32)],
        [_row_spec(ng, 2 * D)],
        tm=tm_g, tk=12288)[0]

    return user_embedding, item_embedding


# speedup vs baseline: 2.5224x; 1.0325x over previous
import functools

import jax
import jax.numpy as jnp
from jax.experimental import pallas as pl
from jax.experimental.pallas import tpu as pltpu

_VMEM_AB = 34 * 1024 * 1024
_VMEM_MAIN = 56 * 1024 * 1024
_GROUPS = 5


def _prelu(y, alpha):
    return jnp.where(y > 0, y, alpha * y)


def _gsum(y):
    tm, _ = y.shape
    ng = tm // _GROUPS
    j = jax.lax.broadcasted_iota(jnp.int32, (ng, tm), 0)
    r = jax.lax.broadcasted_iota(jnp.int32, (ng, tm), 1)
    sel = (r // _GROUPS == j).astype(jnp.float32)
    return jnp.dot(sel, y, preferred_element_type=jnp.float32)


def _dotf(a, b):
    return jnp.dot(a, b, preferred_element_type=jnp.float32)


def _social_body(adj_ref, u_ref, ws_ref, wr_ref, b_ref, alpha_ref,
                 out_ref, ego_ref, sum0_ref, *, tm):
    t = pl.program_id(0)
    ng = tm // _GROUPS

    @pl.when(t < 16)
    def _():
        i = t
        y = _prelu(_dotf(adj_ref[...], u_ref[...]), alpha_ref[...])
        ego_ref[pl.ds(pl.multiple_of(i * tm, tm), tm), :] = _dotf(
            y.astype(jnp.bfloat16), ws_ref[...]).astype(jnp.bfloat16)
        sum0_ref[pl.ds(pl.multiple_of(i * ng, ng), ng), :] = _gsum(y)

    @pl.when(t >= 16)
    def _():
        i = t - 16
        e1 = _prelu(_dotf(adj_ref[...], ego_ref[...]), alpha_ref[...])
        s0 = sum0_ref[pl.ds(pl.multiple_of(i * ng, ng), ng), :]
        s = jnp.concatenate([s0, _gsum(e1)], axis=-1) * 0.25
        out_ref[...] = (_dotf(s.astype(jnp.bfloat16), wr_ref[...])
                        + b_ref[...]).astype(jnp.bfloat16)


def _social_call(adj_s, u_bf16, ws, wr, bias, alpha_row):
    Ms, K = adj_s.shape
    D = u_bf16.shape[1]
    tm = 640
    ng = tm // _GROUPS
    n_users = Ms // _GROUPS
    bcast = lambda shape: pl.BlockSpec(shape, lambda t: (0, 0))
    return pl.pallas_call(
        functools.partial(_social_body, tm=tm),
        out_shape=jax.ShapeDtypeStruct((n_users, D), jnp.bfloat16),
        grid=(32,),
        in_specs=[pl.BlockSpec((tm, K), lambda t: (t % 16, 0)),
                  bcast((Ms, D)), bcast((D, D)), bcast((2 * D, D)),
                  bcast((1, D)), bcast((1, D))],
        out_specs=pl.BlockSpec((ng, D), lambda t: (jnp.clip(t - 16, 0, 15), 0)),
        scratch_shapes=[pltpu.VMEM((Ms, D), jnp.bfloat16),
                        pltpu.VMEM((n_users, D), jnp.float32)],
        compiler_params=pltpu.CompilerParams(
            dimension_semantics=("arbitrary",),
            vmem_limit_bytes=_VMEM_AB),
        cost_estimate=pl.CostEstimate(
            flops=2 * 2 * Ms * K * D, transcendentals=0,
            bytes_accessed=2 * Ms * K * 2),
    )(adj_s, u_bf16, ws, wr, bias, alpha_row)


def _main_body(adju_ref, adji_ref, uv_ref, ie_ref, wu_ref, wi_ref, alpha_ref,
               uout_ref, iout_ref, u0w_ref, i0w_ref, u0_ref, si0_ref,
               *, tmu, tmi, nu):
    t = pl.program_id(0)
    ngi = tmi // _GROUPS
    bf16 = jnp.bfloat16

    def spmm2(adj_ref, left_ref, right_ref):
        y = (_dotf(adj_ref[:, :nu], left_ref[...])
             + _dotf(adj_ref[:, nu:], right_ref[...]))
        return _prelu(y, alpha_ref[...])

    @pl.when(t < 8)
    def _():
        i = t
        y = spmm2(adju_ref, uv_ref, ie_ref)
        u0_ref[pl.ds(pl.multiple_of(i * tmu, tmu), tmu), :] = y
        u0w_ref[pl.ds(pl.multiple_of(i * tmu, tmu), tmu), :] = _dotf(
            y.astype(bf16), wu_ref[...]).astype(bf16)

    @pl.when((t >= 8) & (t < 24))
    def _():
        i = t - 8
        y = spmm2(adji_ref, uv_ref, ie_ref)
        i0w_ref[pl.ds(pl.multiple_of(i * tmi, tmi), tmi), :] = _dotf(
            y.astype(bf16), wi_ref[...]).astype(bf16)
        si0_ref[pl.ds(pl.multiple_of(i * ngi, ngi), ngi), :] = _gsum(y)

    @pl.when((t >= 24) & (t < 32))
    def _():
        i = t - 24
        y = spmm2(adju_ref, u0w_ref, i0w_ref)
        u0 = u0_ref[pl.ds(pl.multiple_of(i * tmu, tmu), tmu), :]
        uout_ref[...] = jnp.concatenate([u0, y], axis=-1)

    @pl.when(t >= 32)
    def _():
        i = t - 32
        y = spmm2(adji_ref, u0w_ref, i0w_ref)
        si0 = si0_ref[pl.ds(pl.multiple_of(i * ngi, ngi), ngi), :]
        iout_ref[...] = jnp.concatenate([si0, _gsum(y)], axis=-1) * 0.2


def _main_call(adj_u, adj_i, user_vec, item_bf16, wu, wi, alpha_row):
    nu, K = adj_u.shape
    Mi = adj_i.shape[0]
    D = user_vec.shape[1]
    tmu, tmi = 256, 640
    ngi = tmi // _GROUPS
    n_items = Mi // _GROUPS
    bcast = lambda shape: pl.BlockSpec(shape, lambda t: (0, 0))

    def adju_map(t):
        return (jnp.where(t < 8, t, jnp.where(t < 24, 0,
                                              jnp.clip(t - 24, 0, 7))), 0)

    def adji_map(t):
        return (jnp.where(t < 24, jnp.clip(t - 8, 0, 15),
                          jnp.clip(t - 32, 0, 15)), 0)

    return pl.pallas_call(
        functools.partial(_main_body, tmu=tmu, tmi=tmi, nu=nu),
        out_shape=(jax.ShapeDtypeStruct((nu, 2 * D), jnp.float32),
                   jax.ShapeDtypeStruct((n_items, 2 * D), jnp.float32)),
        grid=(48,),
        in_specs=[pl.BlockSpec((tmu, K), adju_map),
                  pl.BlockSpec((tmi, K), adji_map),
                  bcast((nu, D)), bcast((Mi, D)), bcast((D, D)),
                  bcast((D, D)), bcast((1, D))],
        out_specs=(
            pl.BlockSpec((tmu, 2 * D), lambda t: (jnp.clip(t - 24, 0, 7), 0)),
            pl.BlockSpec((ngi, 2 * D), lambda t: (jnp.clip(t - 32, 0, 15), 0))),
        scratch_shapes=[pltpu.VMEM((nu, D), jnp.bfloat16),
                        pltpu.VMEM((Mi, D), jnp.bfloat16),
                        pltpu.VMEM((nu, D), jnp.float32),
                        pltpu.VMEM((n_items, D), jnp.float32)],
        compiler_params=pltpu.CompilerParams(
            dimension_semantics=("arbitrary",),
            vmem_limit_bytes=_VMEM_MAIN),
        cost_estimate=pl.CostEstimate(
            flops=2 * 2 * (nu + Mi) * K * D, transcendentals=0,
            bytes_accessed=2 * (nu + Mi) * K * 2),
    )(adj_u, adj_i, user_vec, item_bf16, wu, wi, alpha_row)


def kernel(user_emb, item_emb, social_user_w1, user_w1, item_w1, w_r1_wT,
           w_r1_b, alpha, adj_s, adj_u, adj_i):
    bf16 = jnp.bfloat16
    D = 64
    alpha_row = jnp.broadcast_to(alpha, (1, D)).astype(jnp.float32)

    user_vec = _social_call(adj_s, user_emb.astype(bf16),
                            social_user_w1.astype(bf16),
                            w_r1_wT.astype(bf16), w_r1_b[None, :], alpha_row)
    user_embedding, item_embedding = _main_call(
        adj_u, adj_i, user_vec, item_emb.astype(bf16),
        user_w1.astype(bf16), item_w1.astype(bf16), alpha_row)
    return user_embedding, item_embedding


# speedup vs baseline: 2.8341x; 1.1236x over previous
import jax
import jax.numpy as jnp
from jax.experimental import pallas as pl
from jax.experimental.pallas import tpu as pltpu

_GROUPS = 5
_TMS = 640
_TMU = 512
_NS = 16
_NU = 4
_NI = 16
_B0, _C0, _D0, _E0, _F0, _END = 16, 32, 36, 52, 56, 72


def _prelu(y, alpha):
    return jnp.where(y > 0, y, alpha * y)


def _gsum(y):
    tm, _ = y.shape
    ng = tm // _GROUPS
    j = jax.lax.broadcasted_iota(jnp.int32, (ng, tm), 0)
    r = jax.lax.broadcasted_iota(jnp.int32, (ng, tm), 1)
    sel = (r // _GROUPS == j).astype(jnp.float32)
    return jnp.dot(sel, y, preferred_element_type=jnp.float32)


def _dotf(a, b):
    return jnp.dot(a, b, preferred_element_type=jnp.float32)


def _body(adjs_ref, adju_ref, adji_ref, u_ref, ie_ref, ws_ref, wu_ref,
          wi_ref, wr_ref, b_ref, alpha_ref, uout_ref, iout_ref,
          buf, sem, big_ref, sum0_ref, uv_ref, u0w_ref):
    bf16 = jnp.bfloat16
    nu_cols = 2048

    def copy_s(tile, slot):
        return pltpu.make_async_copy(
            adjs_ref.at[pl.ds(tile * _TMS, _TMS), :],
            buf.at[slot, :, pl.ds(0, 10240)], sem.at[slot])

    def copy_u(tile, slot):
        return pltpu.make_async_copy(
            adju_ref.at[pl.ds(tile * _TMU, _TMU), :],
            buf.at[slot, pl.ds(0, _TMU), :], sem.at[slot])

    def copy_i(tile, slot):
        return pltpu.make_async_copy(
            adji_ref.at[pl.ds(tile * _TMS, _TMS), :],
            buf.at[slot], sem.at[slot])

    def spmm2(a, left, right):
        return _prelu(_dotf(a[:, :nu_cols], left)
                      + _dotf(a[:, nu_cols:], right), alpha_ref[...])

    copy_s(0, 0).start()
    copy_s(1, 1).start()

    def step(t, carry):
        slot = jax.lax.rem(t, 3)
        p = t + 2
        pslot = jax.lax.rem(p, 3)

        @pl.when(p < _C0)
        def _():
            copy_s(jax.lax.rem(p, _NS), pslot).start()

        @pl.when((p >= _C0) & (p < _D0))
        def _():
            copy_u(p - _C0, pslot).start()

        @pl.when((p >= _D0) & (p < _E0))
        def _():
            copy_i(p - _D0, pslot).start()

        @pl.when((p >= _E0) & (p < _F0))
        def _():
            copy_u(p - _E0, pslot).start()

        @pl.when((p >= _F0) & (p < _END))
        def _():
            copy_i(p - _F0, pslot).start()

        @pl.when(t < _B0)
        def _():
            copy_s(t, slot).wait()
            a = buf[slot, :, pl.ds(0, 10240)]
            y = _prelu(_dotf(a, u_ref[...]), alpha_ref[...])
            big_ref[pl.ds(t * _TMS, _TMS), :] = _dotf(
                y.astype(bf16), ws_ref[...]).astype(bf16)
            sum0_ref[pl.ds(t * 128, 128), :] = _gsum(y)

        @pl.when((t >= _B0) & (t < _C0))
        def _():
            i = t - _B0
            copy_s(i, slot).wait()
            a = buf[slot, :, pl.ds(0, 10240)]
            e1 = _prelu(_dotf(a, big_ref[...]), alpha_ref[...])
            s0 = sum0_ref[pl.ds(i * 128, 128), :]
            s = jnp.concatenate([s0, _gsum(e1)], axis=-1) * 0.25
            uv_ref[pl.ds(i * 128, 128), :] = (
                _dotf(s.astype(bf16), wr_ref[...]) + b_ref[...]).astype(bf16)

        @pl.when((t >= _C0) & (t < _D0))
        def _():
            i = t - _C0
            copy_u(i, slot).wait()
            a = buf[slot, pl.ds(0, _TMU), :]
            y = spmm2(a, uv_ref[...], ie_ref[...])
            uout_ref[pl.ds(i * _TMU, _TMU), pl.ds(0, 64)] = y
            u0w_ref[pl.ds(i * _TMU, _TMU), :] = _dotf(
                y.astype(bf16), wu_ref[...]).astype(bf16)

        @pl.when((t >= _D0) & (t < _E0))
        def _():
            i = t - _D0
            copy_i(i, slot).wait()
            y = spmm2(buf[slot], uv_ref[...], ie_ref[...])
            big_ref[pl.ds(i * _TMS, _TMS), :] = _dotf(
                y.astype(bf16), wi_ref[...]).astype(bf16)
            iout_ref[pl.ds(i * 128, 128), pl.ds(0, 64)] = _gsum(y) * 0.2

        @pl.when((t >= _E0) & (t < _F0))
        def _():
            i = t - _E0
            copy_u(i, slot).wait()
            a = buf[slot, pl.ds(0, _TMU), :]
            y = spmm2(a, u0w_ref[...], big_ref[...])
            uout_ref[pl.ds(i * _TMU, _TMU), pl.ds(64, 64)] = y

        @pl.when(t >= _F0)
        def _():
            i = t - _F0
            copy_i(i, slot).wait()
            y = spmm2(buf[slot], u0w_ref[...], big_ref[...])
            iout_ref[pl.ds(i * 128, 128), pl.ds(64, 64)] = _gsum(y) * 0.2

        return carry

    jax.lax.fori_loop(0, _END, step, 0)


def kernel(user_emb, item_emb, social_user_w1, user_w1, item_w1, w_r1_wT,
           w_r1_b, alpha, adj_s, adj_u, adj_i):
    bf16 = jnp.bfloat16
    D = 64
    n_users, n_items = 2048, 2048
    Ms = adj_s.shape[0]
    alpha_row = jnp.broadcast_to(alpha, (1, D)).astype(jnp.float32)

    hbm = pl.BlockSpec(memory_space=pltpu.MemorySpace.HBM)
    vmem = pl.BlockSpec(memory_space=pltpu.MemorySpace.VMEM)

    uout, iout = pl.pallas_call(
        _body,
        out_shape=(jax.ShapeDtypeStruct((n_users, 2 * D), jnp.float32),
                   jax.ShapeDtypeStruct((n_items, 2 * D), jnp.float32)),
        in_specs=[hbm, hbm, hbm] + [vmem] * 8,
        out_specs=(vmem, vmem),
        scratch_shapes=[
            pltpu.VMEM((3, _TMS, 12288), bf16),
            pltpu.SemaphoreType.DMA((3,)),
            pltpu.VMEM((Ms, D), bf16),
            pltpu.VMEM((n_users, D), jnp.float32),
            pltpu.VMEM((n_users, D), bf16),
            pltpu.VMEM((n_users, D), bf16),
        ],
        compiler_params=pltpu.CompilerParams(
            vmem_limit_bytes=60000 * 1024,
            internal_scratch_in_bytes=1024 * 1024),
        cost_estimate=pl.CostEstimate(
            flops=2 * 2 * (Ms * 10240 + (n_users + Ms) * 12288) * D,
            transcendentals=0,
            bytes_accessed=2 * (Ms * 10240 + (n_users + Ms) * 12288) * 2),
    )(adj_s, adj_u, adj_i,
      user_emb.astype(bf16), item_emb.astype(bf16),
      social_user_w1.astype(bf16), user_w1.astype(bf16),
      item_w1.astype(bf16), w_r1_wT.astype(bf16), w_r1_b[None, :],
      alpha_row)
    return uout, iout


# speedup vs baseline: 2.8362x; 1.0007x over previous
import jax
import jax.numpy as jnp
from jax.experimental import pallas as pl
from jax.experimental.pallas import tpu as pltpu

_GROUPS = 5
_TMS = 640
_TMU = 512
_NS = 16
_B0, _C0, _D0, _E0, _F0, _END = 16, 32, 36, 52, 56, 72


def _prelu(y, alpha):
    return jnp.where(y > 0, y, alpha * y)


def _gsum(y):
    tm, _ = y.shape
    ng = tm // _GROUPS
    j = jax.lax.broadcasted_iota(jnp.int32, (ng, tm), 0)
    r = jax.lax.broadcasted_iota(jnp.int32, (ng, tm), 1)
    sel = (r // _GROUPS == j).astype(jnp.float32)
    return jnp.dot(sel, y, preferred_element_type=jnp.float32)


def _dotf(a, b):
    return jnp.dot(a, b, preferred_element_type=jnp.float32)


def _body(adjs_ref, adju_ref, adji_ref, u_ref, ie_ref, ws_ref, wu_ref,
          wi_ref, wr_ref, b_ref, alpha_ref, uout_ref, iout_ref,
          buf, sem, big_ref, sum0_ref, uv_ref, u0w_ref):
    bf16 = jnp.bfloat16
    nu_cols = 2048

    def copy_s(tile, slot):
        return pltpu.make_async_copy(
            adjs_ref.at[pl.ds(tile * _TMS, _TMS), :],
            buf.at[slot, :, pl.ds(0, 10240)], sem.at[slot])

    def copy_u(tile, slot):
        return pltpu.make_async_copy(
            adju_ref.at[pl.ds(tile * _TMU, _TMU), :],
            buf.at[slot, pl.ds(0, _TMU), :], sem.at[slot])

    def copy_i(tile, slot):
        return pltpu.make_async_copy(
            adji_ref.at[pl.ds(tile * _TMS, _TMS), :],
            buf.at[slot], sem.at[slot])

    def spmm2(a, left, right):
        return _prelu(_dotf(a[:, :nu_cols], left)
                      + _dotf(a[:, nu_cols:], right), alpha_ref[...])

    copy_s(0, 0).start()
    copy_s(1, 1).start()

    def step(t, carry):
        slot = jax.lax.rem(t, 3)
        p = t + 2
        pslot = jax.lax.rem(p, 3)

        @pl.when(p < _C0)
        def _():
            copy_s(jax.lax.rem(p, _NS), pslot).start()

        @pl.when((p >= _C0) & (p < _D0))
        def _():
            copy_u(p - _C0, pslot).start()

        @pl.when((p >= _D0) & (p < _E0))
        def _():
            copy_i(p - _D0, pslot).start()

        @pl.when((p >= _E0) & (p < _F0))
        def _():
            copy_u(p - _E0, pslot).start()

        @pl.when((p >= _F0) & (p < _END))
        def _():
            copy_i(p - _F0, pslot).start()

        @pl.when(t < _B0)
        def _():
            copy_s(t, slot).wait()
            a = buf[slot, :, pl.ds(0, 10240)]
            y = _prelu(_dotf(a, u_ref[...]), alpha_ref[...])
            big_ref[pl.ds(t * _TMS, _TMS), :] = _dotf(
                y.astype(bf16), ws_ref[...]).astype(bf16)
            sum0_ref[pl.ds(t * 128, 128), :] = _gsum(y)

        @pl.when((t >= _B0) & (t < _C0))
        def _():
            i = t - _B0
            copy_s(i, slot).wait()
            a = buf[slot, :, pl.ds(0, 10240)]
            e1 = _prelu(_dotf(a, big_ref[...]), alpha_ref[...])
            s0 = sum0_ref[pl.ds(i * 128, 128), :]
            s = jnp.concatenate([s0, _gsum(e1)], axis=-1) * 0.25
            uv_ref[pl.ds(i * 128, 128), :] = (
                _dotf(s.astype(bf16), wr_ref[...]) + b_ref[...]).astype(bf16)

        @pl.when((t >= _C0) & (t < _D0))
        def _():
            i = t - _C0
            copy_u(i, slot).wait()
            a = buf[slot, pl.ds(0, _TMU), :]
            y = spmm2(a, uv_ref[...], ie_ref[...])
            uout_ref[pl.ds(i * _TMU, _TMU), pl.ds(0, 64)] = y
            u0w_ref[pl.ds(i * _TMU, _TMU), :] = _dotf(
                y.astype(bf16), wu_ref[...]).astype(bf16)

        @pl.when((t >= _D0) & (t < _E0))
        def _():
            i = t - _D0
            copy_i(i, slot).wait()
            y = spmm2(buf[slot], uv_ref[...], ie_ref[...])
            big_ref[pl.ds(i * _TMS, _TMS), :] = _dotf(
                y.astype(bf16), wi_ref[...]).astype(bf16)
            iout_ref[pl.ds(i * 128, 128), pl.ds(0, 64)] = _gsum(y) * 0.2

        @pl.when((t >= _E0) & (t < _F0))
        def _():
            i = t - _E0
            copy_u(i, slot).wait()
            a = buf[slot, pl.ds(0, _TMU), :]
            y = spmm2(a, u0w_ref[...], big_ref[...])
            uout_ref[pl.ds(i * _TMU, _TMU), pl.ds(64, 64)] = y

        @pl.when(t >= _F0)
        def _():
            i = t - _F0
            copy_i(i, slot).wait()
            y = spmm2(buf[slot], u0w_ref[...], big_ref[...])
            iout_ref[pl.ds(i * 128, 128), pl.ds(64, 64)] = _gsum(y) * 0.2

        return carry

    jax.lax.fori_loop(0, _END, step, 0)


def kernel(user_emb, item_emb, social_user_w1, user_w1, item_w1, w_r1_wT,
           w_r1_b, alpha, adj_s, adj_u, adj_i):
    bf16 = jnp.bfloat16
    D = 64
    n_users, n_items = 2048, 2048
    Ms = adj_s.shape[0]
    alpha_row = jnp.broadcast_to(alpha, (1, D)).astype(jnp.float32)

    hbm = pl.BlockSpec(memory_space=pltpu.MemorySpace.HBM)
    vmem = pl.BlockSpec(memory_space=pltpu.MemorySpace.VMEM)

    uout, iout = pl.pallas_call(
        _body,
        out_shape=(jax.ShapeDtypeStruct((n_users, 2 * D), jnp.float32),
                   jax.ShapeDtypeStruct((n_items, 2 * D), jnp.float32)),
        in_specs=[hbm, hbm, hbm] + [vmem] * 8,
        out_specs=(vmem, vmem),
        scratch_shapes=[
            pltpu.VMEM((3, _TMS, 12288), bf16),
            pltpu.SemaphoreType.DMA((3,)),
            pltpu.VMEM((Ms, D), bf16),
            pltpu.VMEM((n_users, D), jnp.float32),
            pltpu.VMEM((n_users, D), bf16),
            pltpu.VMEM((n_users, D), bf16),
        ],
        compiler_params=pltpu.CompilerParams(
            vmem_limit_bytes=60000 * 1024,
            internal_scratch_in_bytes=1024 * 1024),
        cost_estimate=pl.CostEstimate(
            flops=2 * 2 * (Ms * 10240 + (n_users + Ms) * 12288) * D,
            transcendentals=0,
            bytes_accessed=2 * (Ms * 10240 + (n_users + Ms) * 12288) * 2),
    )(adj_s, adj_u, adj_i,
      user_emb.astype(bf16), item_emb.astype(bf16),
      social_user_w1.astype(bf16), user_w1.astype(bf16),
      item_w1.astype(bf16), w_r1_wT.astype(bf16), w_r1_b[None, :],
      alpha_row)
    return uout, iout
